# baseline (speedup 1.0000x reference)
import functools

import jax
import jax.numpy as jnp
from jax import lax
from jax.experimental import pallas as pl
from jax.experimental.pallas import tpu as pltpu
from jax.experimental.pallas import tpu_sc as plsc

_N = 100000
_H = 128
_WORDS = _N * _H
_NC = 2
_PER_C = _WORDS // _NC
_CHUNK = 640_000
_NCHUNK = _PER_C // _CHUNK
_NBUF = 3


@functools.partial(
    pl.kernel,
    mesh=plsc.ScalarSubcoreMesh(axis_name="c", num_cores=_NC),
    out_type=jax.ShapeDtypeStruct((_WORDS,), jnp.float32),
    scratch_types=(
        [pltpu.VMEM_SHARED((_CHUNK,), jnp.float32) for _ in range(_NBUF)]
        + [pltpu.SemaphoreType.DMA for _ in range(2 * _NBUF)]
    ),
)
def _sc_copy(tab_hbm, out_hbm, *scratch):
    bufs = scratch[:_NBUF]
    sin = scratch[_NBUF:2 * _NBUF]
    sout = scratch[2 * _NBUF:]
    base = lax.axis_index("c") * _PER_C

    def in_copy(i):
        return pltpu.async_copy(
            tab_hbm.at[pl.ds(base + i * _CHUNK, _CHUNK)],
            bufs[i % _NBUF],
            sin[i % _NBUF],
        )

    def out_copy(i):
        return pltpu.async_copy(
            bufs[i % _NBUF],
            out_hbm.at[pl.ds(base + i * _CHUNK, _CHUNK)],
            sout[i % _NBUF],
        )

    hin = [None] * _NBUF
    hout = {}
    out_waited = set()
    for j in range(min(_NBUF - 1, _NCHUNK)):
        hin[j % _NBUF] = in_copy(j)
    for i in range(_NCHUNK):
        b = i % _NBUF
        hin[b].wait()
        hout[i] = out_copy(i)
        j = i + _NBUF - 1
        if j < _NCHUNK:
            prev = j - _NBUF
            if prev >= 0:
                hout[prev].wait()
                out_waited.add(prev)
            hin[j % _NBUF] = in_copy(j)
    for i in range(_NCHUNK):
        if i not in out_waited:
            hout[i].wait()


def kernel(table):
    flat = table.reshape(_WORDS)
    return _sc_copy(flat).reshape(_N, _H)


# speedup vs baseline: 1.0109x; 1.0109x over previous
import functools

import jax
import jax.numpy as jnp
from jax import lax
from jax.experimental import pallas as pl
from jax.experimental.pallas import tpu as pltpu
from jax.experimental.pallas import tpu_sc as plsc

_N = 100000
_H = 128
_WORDS = _N * _H
_NC = 2
_PER_C = _WORDS // _NC
_CHUNK = 400_000
_NCHUNK = _PER_C // _CHUNK
_NBUF = 4


@functools.partial(
    pl.kernel,
    mesh=plsc.ScalarSubcoreMesh(axis_name="c", num_cores=_NC),
    out_type=jax.ShapeDtypeStruct((_WORDS,), jnp.float32),
    scratch_types=(
        [pltpu.VMEM_SHARED((_CHUNK,), jnp.float32) for _ in range(_NBUF)]
        + [pltpu.SemaphoreType.DMA for _ in range(2 * _NBUF)]
    ),
)
def _sc_copy(tab_hbm, out_hbm, *scratch):
    bufs = scratch[:_NBUF]
    sin = scratch[_NBUF:2 * _NBUF]
    sout = scratch[2 * _NBUF:]
    base = lax.axis_index("c") * _PER_C

    def in_copy(i):
        return pltpu.async_copy(
            tab_hbm.at[pl.ds(base + i * _CHUNK, _CHUNK)],
            bufs[i % _NBUF],
            sin[i % _NBUF],
        )

    def out_copy(i):
        return pltpu.async_copy(
            bufs[i % _NBUF],
            out_hbm.at[pl.ds(base + i * _CHUNK, _CHUNK)],
            sout[i % _NBUF],
        )

    hin = [None] * _NBUF
    hout = {}
    out_waited = set()
    for j in range(min(_NBUF - 1, _NCHUNK)):
        hin[j % _NBUF] = in_copy(j)
    for i in range(_NCHUNK):
        b = i % _NBUF
        hin[b].wait()
        hout[i] = out_copy(i)
        j = i + _NBUF - 1
        if j < _NCHUNK:
            prev = j - _NBUF
            if prev >= 0:
                hout[prev].wait()
                out_waited.add(prev)
            hin[j % _NBUF] = in_copy(j)
    for i in range(_NCHUNK):
        if i not in out_waited:
            hout[i].wait()


def kernel(table):
    flat = table.reshape(_WORDS)
    return _sc_copy(flat).reshape(_N, _H)
